# baseline (speedup 1.0000x reference)
import functools

import jax
import jax.numpy as jnp
from jax import lax
from jax.experimental import pallas as pl
from jax.experimental.pallas import tpu as pltpu
from jax.experimental.pallas import tpu_sc as plsc

N = 10000
NPAD = 10240
E = 320000
EPS = 1e-5
NC = 2
NS = 16
K = 128
ROWS_PER_TILE = NPAD // NS
EP = 327680
CH = EP // K
A_NCH = CH // (NC * NS)
B_NCH = CH // NS

_MESH = plsc.VectorSubcoreMesh(core_axis_name="c", subcore_axis_name="s",
                               num_cores=NC, num_subcores=NS)
_PREC = jax.lax.Precision.HIGHEST


def _zero_fill_gbuf(gbuf):
    cols = gbuf.shape[1] // 16

    def row(i, _):
        def col(k, _):
            gbuf[i, pl.ds(k * 16, 16)] = jnp.zeros((16,), jnp.float32)
            return ()
        lax.fori_loop(0, cols, col, ())
        return ()
    lax.fori_loop(0, 128, row, ())


def _zero_acc_slice(gbuf, acc, sid):
    zbase = sid * ROWS_PER_TILE

    def zloop(m, _):
        pltpu.sync_copy(gbuf, acc.at[pl.ds(zbase + m * 128, 128)])
        return ()
    lax.fori_loop(0, ROWS_PER_TILE // 128, zloop, ())


BLK = 16


def _agg_edges(table, acc, sidx2d, didx2d, g0, g1, sem0, sem1):
    pltpu.async_copy(table.at[sidx2d.at[0]], g0, sem0)

    def pair(p, _):
        pltpu.async_copy(table.at[sidx2d.at[2 * p + 1]], g1, sem1)
        pltpu.make_async_copy(table.at[sidx2d.at[2 * p]], g0, sem0).wait()
        pltpu.sync_copy(g0, acc.at[didx2d.at[2 * p]], add=True)
        pltpu.async_copy(table.at[sidx2d.at[2 * p + 2]], g0, sem0)
        pltpu.make_async_copy(table.at[sidx2d.at[2 * p + 1]], g1, sem1).wait()
        pltpu.sync_copy(g1, acc.at[didx2d.at[2 * p + 1]], add=True)
        return ()
    lax.fori_loop(0, BLK // 2 - 1, pair, ())

    pltpu.async_copy(table.at[sidx2d.at[BLK - 1]], g1, sem1)
    pltpu.make_async_copy(table.at[sidx2d.at[BLK - 2]], g0, sem0).wait()
    pltpu.sync_copy(g0, acc.at[didx2d.at[BLK - 2]], add=True)
    pltpu.make_async_copy(table.at[sidx2d.at[BLK - 1]], g1, sem1).wait()
    pltpu.sync_copy(g1, acc.at[didx2d.at[BLK - 1]], add=True)


def _agg_run(src2d, dst2d, table, acc, sidx2d, didx2d, g0, g1,
             sem0, sem1, cbase, nch):
    assert nch % BLK == 0

    def block(b, _):
        base = cbase + b * BLK
        pltpu.sync_copy(src2d.at[pl.ds(base, BLK)], sidx2d)
        pltpu.sync_copy(dst2d.at[pl.ds(base, BLK)], didx2d)
        _agg_edges(table, acc, sidx2d, didx2d, g0, g1, sem0, sem1)
        return ()
    lax.fori_loop(0, nch // BLK, block, ())


@functools.partial(
    pl.kernel,
    out_type=jax.ShapeDtypeStruct((NC, NPAD, 128), jnp.float32),
    mesh=_MESH,
    scratch_types=[
        pltpu.VMEM((16, K), jnp.int32),
        pltpu.VMEM((K, 128), jnp.float32),
        pltpu.VMEM((K, 128), jnp.float32),
        pltpu.SemaphoreType.DMA,
        pltpu.VMEM_SHARED((NPAD, 128), jnp.float32),
    ],
)
def _deg_kernel(dst2d_hbm, out_hbm, didx2d, ones, zbuf, sem, acc):
    cid = lax.axis_index("c")
    sid = lax.axis_index("s")
    wid = cid * NS + sid

    def fill(i, _):
        def fc(j, _):
            ones[i, pl.ds(j * 16, 16)] = jnp.ones((16,), jnp.float32)
            zbuf[i, pl.ds(j * 16, 16)] = jnp.zeros((16,), jnp.float32)
            return ()
        lax.fori_loop(0, 8, fc, ())
        return ()
    lax.fori_loop(0, K, fill, ())
    _zero_acc_slice(zbuf, acc, sid)
    plsc.subcore_barrier()

    def block(b, _):
        pltpu.sync_copy(dst2d_hbm.at[pl.ds(wid * A_NCH + b * 16, 16)], didx2d)

        def chunk(j, _):
            pltpu.async_copy(ones, acc.at[didx2d.at[j]], sem, add=True)
            return ()
        lax.fori_loop(0, 16, chunk, ())

        def drain(j, _):
            pltpu.make_async_copy(ones, acc.at[didx2d.at[j]], sem).wait()
            return ()
        lax.fori_loop(0, 16, drain, ())
        return ()
    lax.fori_loop(0, A_NCH // 16, block, ())

    plsc.subcore_barrier()
    rbase = sid * ROWS_PER_TILE

    @pl.when(cid == 0)
    def _():
        pltpu.sync_copy(acc.at[pl.ds(rbase, ROWS_PER_TILE)],
                        out_hbm.at[0, pl.ds(rbase, ROWS_PER_TILE)])

    @pl.when(cid == 1)
    def _():
        pltpu.sync_copy(acc.at[pl.ds(rbase, ROWS_PER_TILE)],
                        out_hbm.at[1, pl.ds(rbase, ROWS_PER_TILE)])


@functools.partial(
    pl.kernel,
    out_type=jax.ShapeDtypeStruct((NC, NPAD, 128), jnp.float32),
    mesh=_MESH,
    scratch_types=[
        pltpu.VMEM((BLK, K), jnp.int32),
        pltpu.VMEM((BLK, K), jnp.int32),
        pltpu.VMEM((K, 128), jnp.float32),
        pltpu.VMEM((K, 128), jnp.float32),
        pltpu.SemaphoreType.DMA,
        pltpu.SemaphoreType.DMA,
        pltpu.VMEM_SHARED((NPAD, 128), jnp.float32),
    ],
)
def _agg_a_kernel(src2d_hbm, dst2d_hbm, table_hbm, out_hbm,
                  sidx2d, didx2d, g0, g1, sem0, sem1, acc):
    cid = lax.axis_index("c")
    sid = lax.axis_index("s")
    wid = cid * NS + sid

    _zero_fill_gbuf(g0)
    _zero_acc_slice(g0, acc, sid)
    plsc.subcore_barrier()

    _agg_run(src2d_hbm, dst2d_hbm, table_hbm, acc, sidx2d, didx2d,
             g0, g1, sem0, sem1, wid * A_NCH, A_NCH)

    plsc.subcore_barrier()
    rbase = sid * ROWS_PER_TILE

    @pl.when(cid == 0)
    def _():
        pltpu.sync_copy(acc.at[pl.ds(rbase, ROWS_PER_TILE)],
                        out_hbm.at[0, pl.ds(rbase, ROWS_PER_TILE)])

    @pl.when(cid == 1)
    def _():
        pltpu.sync_copy(acc.at[pl.ds(rbase, ROWS_PER_TILE)],
                        out_hbm.at[1, pl.ds(rbase, ROWS_PER_TILE)])


@functools.partial(
    pl.kernel,
    out_type=(jax.ShapeDtypeStruct((NPAD, 128), jnp.float32),
              jax.ShapeDtypeStruct((NPAD, 128), jnp.float32)),
    mesh=_MESH,
    scratch_types=[
        pltpu.VMEM((BLK, K), jnp.int32),
        pltpu.VMEM((BLK, K), jnp.int32),
        pltpu.VMEM((K, 128), jnp.float32),
        pltpu.VMEM((K, 128), jnp.float32),
        pltpu.SemaphoreType.DMA,
        pltpu.SemaphoreType.DMA,
        pltpu.VMEM_SHARED((NPAD, 128), jnp.float32),
    ],
)
def _agg_b_kernel(src2d_hbm, dst2d_hbm, ta_hbm, tb_hbm, outa_hbm, outb_hbm,
                  sidx2d, didx2d, g0, g1, sem0, sem1, acc):
    cid = lax.axis_index("c")
    sid = lax.axis_index("s")

    _zero_fill_gbuf(g0)
    _zero_acc_slice(g0, acc, sid)
    plsc.subcore_barrier()

    cbase = sid * B_NCH

    @pl.when(cid == 0)
    def _():
        _agg_run(src2d_hbm, dst2d_hbm, ta_hbm, acc, sidx2d, didx2d,
                 g0, g1, sem0, sem1, cbase, B_NCH)

    @pl.when(cid == 1)
    def _():
        _agg_run(src2d_hbm, dst2d_hbm, tb_hbm, acc, sidx2d, didx2d,
                 g0, g1, sem0, sem1, cbase, B_NCH)

    plsc.subcore_barrier()
    rbase = sid * ROWS_PER_TILE

    @pl.when(cid == 0)
    def _():
        pltpu.sync_copy(acc.at[pl.ds(rbase, ROWS_PER_TILE)],
                        outa_hbm.at[pl.ds(rbase, ROWS_PER_TILE)])

    @pl.when(cid == 1)
    def _():
        pltpu.sync_copy(acc.at[pl.ds(rbase, ROWS_PER_TILE)],
                        outb_hbm.at[pl.ds(rbase, ROWS_PER_TILE)])


def _dot(a, b):
    return jnp.dot(a, b, precision=_PREC, preferred_element_type=jnp.float32)


def _row_mask():
    return (lax.broadcasted_iota(jnp.int32, (NPAD, 1), 0) < N).astype(jnp.float32)


def _bn_relu(h, g, be, relu):
    mask = _row_mask()
    h = h * mask
    mean = jnp.sum(h, axis=0, keepdims=True) * (1.0 / N)
    d = (h - mean) * mask
    var = jnp.sum(d * d, axis=0, keepdims=True) * (1.0 / N)
    hn = (h - mean) * lax.rsqrt(var + EPS) * g + be
    if relu:
        hn = jnp.maximum(hn, 0.0)
    return hn * mask


def _tc1_body(x_ref, w_ref, degp_ref, xws_ref, dinv_ref):
    deg = degp_ref[0] + degp_ref[1] + 1.0
    dinv = lax.rsqrt(deg)[:, 0:1]
    xw = _dot(x_ref[...], w_ref[...])
    xws_ref[...] = xw * dinv
    dinv_ref[...] = dinv


def _tc_post_body(agg_ref, xws_ref, dinv_ref, b_ref, g_ref, be_ref, out_ref,
                  *, relu):
    h = (dinv_ref[...] * (agg_ref[0] + agg_ref[1] + xws_ref[...])
         + b_ref[...])
    out_ref[...] = _bn_relu(h, g_ref[...], be_ref[...], relu=relu)


def _tc_post_final_body(agg_ref, xws_ref, dinv_ref, b_ref, g_ref, be_ref,
                        out_ref):
    h = (dinv_ref[...] * (agg_ref[0] + agg_ref[1] + xws_ref[...])
         + b_ref[...])
    hn = _bn_relu(h, g_ref[...], be_ref[...], relu=False)
    out_ref[...] = hn[0:N, :]


def _tc_post_half_body(agg_ref, xws_ref, dinv_ref, b_ref, g_ref, be_ref,
                       out_ref):
    h = dinv_ref[...] * (agg_ref[...] + xws_ref[...]) + b_ref[...]
    out_ref[...] = _bn_relu(h, g_ref[...], be_ref[...], relu=True)


def _tc_pre256_body(hn_ref, w_ref, dinv_ref, outa_ref, outb_ref):
    xw = _dot(hn_ref[...], w_ref[...]) * dinv_ref[...]
    outa_ref[...] = xw[:, 0:128]
    outb_ref[...] = xw[:, 128:256]


def _tc_pre_cat_body(hna_ref, hnb_ref, w_ref, dinv_ref, out_ref):
    xw = (_dot(hna_ref[...], w_ref[0:128, :])
          + _dot(hnb_ref[...], w_ref[128:256, :]))
    out_ref[...] = xw * dinv_ref[...]


_F32 = jnp.float32
_P128 = jax.ShapeDtypeStruct((NPAD, 128), _F32)
_CP = pltpu.CompilerParams(vmem_limit_bytes=128 * 1024 * 1024)
_tc1 = pl.pallas_call(
    _tc1_body, compiler_params=_CP,
    out_shape=(_P128, jax.ShapeDtypeStruct((NPAD, 1), _F32)))
_tc_post = pl.pallas_call(
    functools.partial(_tc_post_body, relu=True), compiler_params=_CP,
    out_shape=_P128)
_tc_post_final = pl.pallas_call(
    _tc_post_final_body, compiler_params=_CP,
    out_shape=jax.ShapeDtypeStruct((N, 128), _F32))
_tc_post_half = pl.pallas_call(_tc_post_half_body, compiler_params=_CP,
                               out_shape=_P128)
_tc_pre256 = pl.pallas_call(_tc_pre256_body, compiler_params=_CP,
                            out_shape=(_P128, _P128))
_tc_pre_cat = pl.pallas_call(_tc_pre_cat_body, compiler_params=_CP,
                             out_shape=_P128)


def kernel(x, edge_index, W1, b1, g1, be1, W2, b2, g2, be2, W5, b5, g5, be5):
    pad = jnp.full((EP - E,), N, jnp.int32)
    src2d = jnp.concatenate([edge_index[0], pad]).reshape(CH, K)
    dst2d = jnp.concatenate([edge_index[1], pad]).reshape(CH, K)
    xp = jnp.zeros((NPAD, x.shape[1]), jnp.float32).at[:N].set(x)
    r = lambda v: v.reshape(1, -1)

    degp = _deg_kernel(dst2d)
    xws1, dinv = _tc1(xp, W1, degp)
    agg1 = _agg_a_kernel(src2d, dst2d, xws1)
    hn1 = _tc_post(agg1, xws1, dinv, r(b1), r(g1), r(be1))
    xws2a, xws2b = _tc_pre256(hn1, W2, dinv)
    agg2a, agg2b = _agg_b_kernel(src2d, dst2d, xws2a, xws2b)
    hn2a = _tc_post_half(agg2a, xws2a, dinv,
                         r(b2[0:128]), r(g2[0:128]), r(be2[0:128]))
    hn2b = _tc_post_half(agg2b, xws2b, dinv,
                         r(b2[128:256]), r(g2[128:256]), r(be2[128:256]))
    xws5 = _tc_pre_cat(hn2a, hn2b, W5, dinv)
    agg3 = _agg_a_kernel(src2d, dst2d, xws5)
    out = _tc_post_final(agg3, xws5, dinv, r(b5), r(g5), r(be5))
    return out


# speedup vs baseline: 2.0489x; 2.0489x over previous
import functools

import jax
import jax.numpy as jnp
from jax import lax
from jax.experimental import pallas as pl
from jax.experimental.pallas import tpu as pltpu
from jax.experimental.pallas import tpu_sc as plsc

N = 10000
NPAD = 10240
E = 320000
EPS = 1e-5
NC = 2
NS = 16
K = 128
ROWS_PER_TILE = NPAD // NS

_MESH = plsc.VectorSubcoreMesh(core_axis_name="c", subcore_axis_name="s",
                               num_cores=NC, num_subcores=NS)
_PREC = jax.lax.Precision.HIGHEST


def _zero_fill_gbuf(gbuf):
    cols = gbuf.shape[1] // 16

    def row(i, _):
        def col(k, _):
            gbuf[i, pl.ds(k * 16, 16)] = jnp.zeros((16,), jnp.float32)
            return ()
        lax.fori_loop(0, cols, col, ())
        return ()
    lax.fori_loop(0, 128, row, ())


def _zero_acc_slice(gbuf, acc, sid):
    zbase = sid * ROWS_PER_TILE

    def zloop(m, _):
        pltpu.sync_copy(gbuf, acc.at[pl.ds(zbase + m * 128, 128)])
        return ()
    lax.fori_loop(0, ROWS_PER_TILE // 128, zloop, ())


def _agg_edges(src_hbm, dst_hbm, table, acc, s0, d0, g0, s1, d1, g1,
               sidx_r, didx_r, gbuf_r, sem0, sem1, ebase, n_full, rem):
    assert n_full % 2 == 0 and n_full >= 2
    pairs = n_full // 2

    def load(off, s, d):
        pltpu.sync_copy(src_hbm.at[pl.ds(off, K)], s)
        pltpu.sync_copy(dst_hbm.at[pl.ds(off, K)], d)

    load(ebase, s0, d0)
    pltpu.async_copy(table.at[s0], g0, sem0)

    def pair(p, _):
        load(ebase + (2 * p + 1) * K, s1, d1)
        pltpu.async_copy(table.at[s1], g1, sem1)
        pltpu.make_async_copy(table.at[s0], g0, sem0).wait()
        pltpu.sync_copy(g0, acc.at[d0], add=True)
        load(ebase + (2 * p + 2) * K, s0, d0)
        pltpu.async_copy(table.at[s0], g0, sem0)
        pltpu.make_async_copy(table.at[s1], g1, sem1).wait()
        pltpu.sync_copy(g1, acc.at[d1], add=True)
        return ()
    lax.fori_loop(0, pairs - 1, pair, ())

    load(ebase + (n_full - 1) * K, s1, d1)
    pltpu.async_copy(table.at[s1], g1, sem1)
    pltpu.make_async_copy(table.at[s0], g0, sem0).wait()
    pltpu.sync_copy(g0, acc.at[d0], add=True)
    pltpu.make_async_copy(table.at[s1], g1, sem1).wait()
    pltpu.sync_copy(g1, acc.at[d1], add=True)

    off = ebase + n_full * K
    pltpu.sync_copy(src_hbm.at[pl.ds(off, rem)], sidx_r)
    pltpu.sync_copy(dst_hbm.at[pl.ds(off, rem)], didx_r)
    pltpu.async_copy(table.at[sidx_r], gbuf_r, sem0).wait()
    pltpu.sync_copy(gbuf_r, acc.at[didx_r], add=True)


@functools.partial(
    pl.kernel,
    out_type=jax.ShapeDtypeStruct((NC, NPAD, 128), jnp.float32),
    mesh=_MESH,
    scratch_types=[
        pltpu.VMEM((K,), jnp.int32),
        pltpu.VMEM((16,), jnp.int32),
        pltpu.VMEM((K, 128), jnp.float32),
        pltpu.VMEM((K, 128), jnp.float32),
        pltpu.VMEM_SHARED((NPAD, 128), jnp.float32),
    ],
)
def _deg_kernel(dst_hbm, out_hbm, didx, didx_r, ones, zbuf, acc):
    cid = lax.axis_index("c")
    sid = lax.axis_index("s")
    wid = cid * NS + sid

    def fill(i, _):
        def fc(j, _):
            ones[i, pl.ds(j * 16, 16)] = jnp.ones((16,), jnp.float32)
            zbuf[i, pl.ds(j * 16, 16)] = jnp.zeros((16,), jnp.float32)
            return ()
        lax.fori_loop(0, 8, fc, ())
        return ()
    lax.fori_loop(0, K, fill, ())
    _zero_acc_slice(zbuf, acc, sid)
    plsc.subcore_barrier()

    e_per = E // (NC * NS)
    n_full, rem = e_per // K, e_per % K
    ebase = wid * e_per

    def chunk(j, _):
        pltpu.sync_copy(dst_hbm.at[pl.ds(ebase + j * K, K)], didx)
        pltpu.sync_copy(ones, acc.at[didx], add=True)
        return ()
    lax.fori_loop(0, n_full, chunk, ())
    pltpu.sync_copy(dst_hbm.at[pl.ds(ebase + n_full * K, rem)], didx_r)
    pltpu.sync_copy(ones.at[pl.ds(0, rem)], acc.at[didx_r], add=True)

    plsc.subcore_barrier()
    rbase = sid * ROWS_PER_TILE

    @pl.when(cid == 0)
    def _():
        pltpu.sync_copy(acc.at[pl.ds(rbase, ROWS_PER_TILE)],
                        out_hbm.at[0, pl.ds(rbase, ROWS_PER_TILE)])

    @pl.when(cid == 1)
    def _():
        pltpu.sync_copy(acc.at[pl.ds(rbase, ROWS_PER_TILE)],
                        out_hbm.at[1, pl.ds(rbase, ROWS_PER_TILE)])


@functools.partial(
    pl.kernel,
    out_type=jax.ShapeDtypeStruct((NC, NPAD, 128), jnp.float32),
    mesh=_MESH,
    scratch_types=[
        pltpu.VMEM((K,), jnp.int32),
        pltpu.VMEM((K,), jnp.int32),
        pltpu.VMEM((K, 128), jnp.float32),
        pltpu.VMEM((K,), jnp.int32),
        pltpu.VMEM((K,), jnp.int32),
        pltpu.VMEM((K, 128), jnp.float32),
        pltpu.VMEM((16,), jnp.int32),
        pltpu.VMEM((16,), jnp.int32),
        pltpu.VMEM((16, 128), jnp.float32),
        pltpu.SemaphoreType.DMA,
        pltpu.SemaphoreType.DMA,
        pltpu.VMEM_SHARED((NPAD, 128), jnp.float32),
    ],
)
def _agg_a_kernel(src_hbm, dst_hbm, table_hbm, out_hbm,
                  s0, d0, g0, s1, d1, g1, sidx_r, didx_r, gbuf_r,
                  sem0, sem1, acc):
    cid = lax.axis_index("c")
    sid = lax.axis_index("s")

    _zero_fill_gbuf(g0)
    _zero_acc_slice(g0, acc, sid)
    plsc.subcore_barrier()

    e_per = E // (NC * NS)
    ebase = cid * (E // NC) + sid * e_per
    _agg_edges(src_hbm, dst_hbm, table_hbm, acc, s0, d0, g0, s1, d1, g1,
               sidx_r, didx_r, gbuf_r, sem0, sem1, ebase, e_per // K, e_per % K)

    plsc.subcore_barrier()
    rbase = sid * ROWS_PER_TILE

    @pl.when(cid == 0)
    def _():
        pltpu.sync_copy(acc.at[pl.ds(rbase, ROWS_PER_TILE)],
                        out_hbm.at[0, pl.ds(rbase, ROWS_PER_TILE)])

    @pl.when(cid == 1)
    def _():
        pltpu.sync_copy(acc.at[pl.ds(rbase, ROWS_PER_TILE)],
                        out_hbm.at[1, pl.ds(rbase, ROWS_PER_TILE)])


@functools.partial(
    pl.kernel,
    out_type=(jax.ShapeDtypeStruct((NPAD, 128), jnp.float32),
              jax.ShapeDtypeStruct((NPAD, 128), jnp.float32)),
    mesh=_MESH,
    scratch_types=[
        pltpu.VMEM((K,), jnp.int32),
        pltpu.VMEM((K,), jnp.int32),
        pltpu.VMEM((K, 128), jnp.float32),
        pltpu.VMEM((K,), jnp.int32),
        pltpu.VMEM((K,), jnp.int32),
        pltpu.VMEM((K, 128), jnp.float32),
        pltpu.VMEM((32,), jnp.int32),
        pltpu.VMEM((32,), jnp.int32),
        pltpu.VMEM((32, 128), jnp.float32),
        pltpu.SemaphoreType.DMA,
        pltpu.SemaphoreType.DMA,
        pltpu.VMEM_SHARED((NPAD, 128), jnp.float32),
    ],
)
def _agg_b_kernel(src_hbm, dst_hbm, ta_hbm, tb_hbm, outa_hbm, outb_hbm,
                  s0, d0, g0, s1, d1, g1, sidx_r, didx_r, gbuf_r,
                  sem0, sem1, acc):
    cid = lax.axis_index("c")
    sid = lax.axis_index("s")

    _zero_fill_gbuf(g0)
    _zero_acc_slice(g0, acc, sid)
    plsc.subcore_barrier()

    e_per = E // NS
    ebase = sid * e_per
    n_full, rem = e_per // K, e_per % K

    @pl.when(cid == 0)
    def _():
        _agg_edges(src_hbm, dst_hbm, ta_hbm, acc, s0, d0, g0, s1, d1, g1,
                   sidx_r, didx_r, gbuf_r, sem0, sem1, ebase, n_full, rem)

    @pl.when(cid == 1)
    def _():
        _agg_edges(src_hbm, dst_hbm, tb_hbm, acc, s0, d0, g0, s1, d1, g1,
                   sidx_r, didx_r, gbuf_r, sem0, sem1, ebase, n_full, rem)

    plsc.subcore_barrier()
    rbase = sid * ROWS_PER_TILE

    @pl.when(cid == 0)
    def _():
        pltpu.sync_copy(acc.at[pl.ds(rbase, ROWS_PER_TILE)],
                        outa_hbm.at[pl.ds(rbase, ROWS_PER_TILE)])

    @pl.when(cid == 1)
    def _():
        pltpu.sync_copy(acc.at[pl.ds(rbase, ROWS_PER_TILE)],
                        outb_hbm.at[pl.ds(rbase, ROWS_PER_TILE)])


def _dot(a, b):
    return jnp.dot(a, b, precision=_PREC, preferred_element_type=jnp.float32)


def _row_mask():
    return (lax.broadcasted_iota(jnp.int32, (NPAD, 1), 0) < N).astype(jnp.float32)


def _bn_relu(h, g, be, relu):
    mask = _row_mask()
    h = h * mask
    mean = jnp.sum(h, axis=0, keepdims=True) * (1.0 / N)
    d = (h - mean) * mask
    var = jnp.sum(d * d, axis=0, keepdims=True) * (1.0 / N)
    hn = (h - mean) * lax.rsqrt(var + EPS) * g + be
    if relu:
        hn = jnp.maximum(hn, 0.0)
    return hn * mask


def _tc1_body(x_ref, w_ref, degp_ref, xws_ref, dinv_ref):
    deg = degp_ref[0] + degp_ref[1] + 1.0
    dinv = lax.rsqrt(deg)[:, 0:1]
    xw = _dot(x_ref[...], w_ref[...])
    xws_ref[...] = xw * dinv
    dinv_ref[...] = dinv


def _tc_post_final_body(agg_ref, xws_ref, dinv_ref, b_ref, g_ref, be_ref,
                        out_ref):
    h = (dinv_ref[...] * (agg_ref[0] + agg_ref[1] + xws_ref[...])
         + b_ref[...])
    hn = _bn_relu(h, g_ref[...], be_ref[...], relu=False)
    out_ref[...] = hn[0:N, :]


def _tc_post_body(agg_ref, xws_ref, dinv_ref, b_ref, g_ref, be_ref, out_ref,
                  *, relu):
    h = (dinv_ref[...] * (agg_ref[0] + agg_ref[1] + xws_ref[...])
         + b_ref[...])
    out_ref[...] = _bn_relu(h, g_ref[...], be_ref[...], relu=relu)


def _tc_pre256_body(hn_ref, w_ref, dinv_ref, outa_ref, outb_ref):
    xw = _dot(hn_ref[...], w_ref[...]) * dinv_ref[...]
    outa_ref[...] = xw[:, 0:128]
    outb_ref[...] = xw[:, 128:256]


def _tc_post_half_body(agg_ref, xws_ref, dinv_ref, b_ref, g_ref, be_ref,
                       out_ref):
    h = dinv_ref[...] * (agg_ref[...] + xws_ref[...]) + b_ref[...]
    out_ref[...] = _bn_relu(h, g_ref[...], be_ref[...], relu=True)


def _tc_pre_cat_body(hna_ref, hnb_ref, w_ref, dinv_ref, out_ref):
    xw = (_dot(hna_ref[...], w_ref[0:128, :])
          + _dot(hnb_ref[...], w_ref[128:256, :]))
    out_ref[...] = xw * dinv_ref[...]


_F32 = jnp.float32
_P128 = jax.ShapeDtypeStruct((NPAD, 128), _F32)
_CP = pltpu.CompilerParams(vmem_limit_bytes=128 * 1024 * 1024)
_tc1 = pl.pallas_call(
    _tc1_body, compiler_params=_CP,
    out_shape=(_P128, jax.ShapeDtypeStruct((NPAD, 1), _F32)))
_tc_post_final = pl.pallas_call(
    _tc_post_final_body, compiler_params=_CP,
    out_shape=jax.ShapeDtypeStruct((N, 128), _F32))
_tc_post = pl.pallas_call(
    functools.partial(_tc_post_body, relu=True), compiler_params=_CP,
    out_shape=_P128)
_tc_pre256 = pl.pallas_call(_tc_pre256_body, compiler_params=_CP,
                            out_shape=(_P128, _P128))
_tc_post_half = pl.pallas_call(_tc_post_half_body, compiler_params=_CP,
                               out_shape=_P128)
_tc_pre_cat = pl.pallas_call(_tc_pre_cat_body, compiler_params=_CP,
                             out_shape=_P128)


def kernel(x, edge_index, W1, b1, g1, be1, W2, b2, g2, be2, W5, b5, g5, be5):
    src = edge_index[0]
    dst = edge_index[1]
    xp = jnp.zeros((NPAD, x.shape[1]), jnp.float32).at[:N].set(x)
    r = lambda v: v.reshape(1, -1)

    degp = _deg_kernel(dst)
    xws1, dinv = _tc1(xp, W1, degp)
    agg1 = _agg_a_kernel(src, dst, xws1)
    hn1 = _tc_post(agg1, xws1, dinv, r(b1), r(g1), r(be1))
    xws2a, xws2b = _tc_pre256(hn1, W2, dinv)
    agg2a, agg2b = _agg_b_kernel(src, dst, xws2a, xws2b)
    hn2a = _tc_post_half(agg2a, xws2a, dinv,
                         r(b2[0:128]), r(g2[0:128]), r(be2[0:128]))
    hn2b = _tc_post_half(agg2b, xws2b, dinv,
                         r(b2[128:256]), r(g2[128:256]), r(be2[128:256]))
    xws5 = _tc_pre_cat(hn2a, hn2b, W5, dinv)
    agg3 = _agg_a_kernel(src, dst, xws5)
    out = _tc_post_final(agg3, xws5, dinv, r(b5), r(g5), r(be5))
    return out
